# baseline (speedup 1.0000x reference)
import functools
import jax
import jax.numpy as jnp
from jax import lax
from jax.experimental import pallas as pl
from jax.experimental.pallas import tpu as pltpu
from jax.experimental.pallas import tpu_sc as plsc

_ROWS = 128
_V = 100000
_K = 64
_F = 2000
_C = 50
_CAP = 4096
_UCH = 10000
_RPW = 4
_RB = 8
_NEG = -3.4e38
_IMIN = -2147483648
_MANT = 0x7FFFFFFF


def _sc_body(logits_hbm, u_hbm, oval, oidx, ou,
             row_v, uch_v, kcm_v, cval_v, cidx_v, cu_v):
    wid = lax.axis_index("s") * 2 + lax.axis_index("c")
    lane = lax.iota(jnp.int32, 16)
    for rr in range(_RPW):
        r = wid * _RPW + rr
        pltpu.sync_copy(logits_hbm.at[r], row_v)

        def fill(i, _):
            sl = pl.ds(i * 16, 16)
            cval_v[sl] = jnp.full((16,), _NEG, jnp.float32)
            cu_v[sl] = jnp.full((16,), jnp.float32(0.5))
            cidx_v[sl] = jnp.full((16,), r * _V, jnp.int32)
            return 0
        lax.fori_loop(0, _CAP // 16, fill, 0)

        def fold(j, _):
            base = j * 16
            acc = row_v[pl.ds(base, 16)]
            for c in range(1, _C):
                acc = jnp.maximum(acc, row_v[pl.ds(c * _F + base, 16)])
            bits = plsc.bitcast(acc, jnp.int32)
            kcm_v[pl.ds(base, 16)] = jnp.where(bits < 0, bits ^ _MANT, bits)
            return 0
        lax.fori_loop(0, _F // 16, fold, 0)

        def count_ge(t):
            def cnt(j, acc):
                k16 = kcm_v[pl.ds(j * 16, 16)]
                return acc + jnp.where(k16 >= t, jnp.int32(1), jnp.int32(0))
            acc = lax.fori_loop(0, _F // 16, cnt, jnp.zeros((16,), jnp.int32))
            return jnp.sum(acc)

        kf = jnp.int32(_K)
        base0 = jnp.where(count_ge(jnp.int32(0)) >= kf, jnp.int32(0),
                          jnp.int32(_IMIN))

        def bit(i, b):
            cand = b | (jnp.int32(1) << (jnp.int32(30) - i))
            return jnp.where(count_ge(cand) >= kf, cand, b)
        t0k = lax.fori_loop(0, 31, bit, base0)

        tkv = jnp.full((16,), t0k, jnp.int32)
        t0f = plsc.bitcast(jnp.where(tkv < 0, tkv ^ _MANT, tkv), jnp.float32)

        cursor = jnp.int32(0)
        for c in range(_V // _UCH):
            pltpu.sync_copy(u_hbm.at[r, pl.ds(c * _UCH, _UCH)], uch_v)

            def scan(j, cur):
                x16 = row_v[pl.ds(c * _UCH + j * 16, 16)]
                m = x16 >= t0f
                npass = jnp.max(plsc.all_reduce_population_count(m))
                off = jnp.minimum(cur, jnp.int32(_CAP - 16))
                plsc.store_compressed(cval_v.at[pl.ds(off, 16)], x16, mask=m)
                u16 = uch_v[pl.ds(j * 16, 16)]
                plsc.store_compressed(cu_v.at[pl.ds(off, 16)], u16, mask=m)
                gi = lane + (r * _V + c * _UCH + j * 16)
                plsc.store_compressed(cidx_v.at[pl.ds(off, 16)], gi, mask=m)
                return cur + npass
            cursor = lax.fori_loop(0, _UCH // 16, scan, cursor)

        pltpu.sync_copy(cval_v, oval.at[r])
        pltpu.sync_copy(cidx_v, oidx.at[r])
        pltpu.sync_copy(cu_v, ou.at[r])
    plsc.subcore_barrier()


def _sc_filter(logits, u):
    mesh = plsc.VectorSubcoreMesh(core_axis_name="c", subcore_axis_name="s",
                                  num_cores=2, num_subcores=16)
    return pl.kernel(
        _sc_body,
        out_type=[
            jax.ShapeDtypeStruct((_ROWS, _CAP), jnp.float32),
            jax.ShapeDtypeStruct((_ROWS, _CAP), jnp.int32),
            jax.ShapeDtypeStruct((_ROWS, _CAP), jnp.float32),
        ],
        mesh=mesh,
        scratch_types=[
            pltpu.VMEM((_V,), jnp.float32),
            pltpu.VMEM((_UCH,), jnp.float32),
            pltpu.VMEM((_F,), jnp.int32),
            pltpu.VMEM((_CAP,), jnp.float32),
            pltpu.VMEM((_CAP,), jnp.int32),
            pltpu.VMEM((_CAP,), jnp.float32),
        ],
        compiler_params=pltpu.CompilerParams(use_tc_tiling_on_sc=False,
                                             needs_layout_passes=False,
                                             has_side_effects=True),
    )(logits, u)


def _tc_body(val_ref, idx_ref, u_ref, samp_ref, conf_ref):
    x = val_ref[...]
    bits = lax.bitcast_convert_type(x, jnp.int32)
    key = jnp.where(bits < 0, bits ^ _MANT, bits)
    kf = jnp.float32(_K)

    def count_ge(t):
        return jnp.sum((key >= t).astype(jnp.float32), axis=1, keepdims=True)

    neg = jnp.full((_RB, 1), _IMIN, jnp.int32)
    zero = jnp.zeros((_RB, 1), jnp.int32)
    base = jnp.where(count_ge(zero) >= kf, zero, neg)
    for b in range(30, -1, -1):
        cand = base | jnp.int32(1 << b)
        base = jnp.where(count_ge(cand) >= kf, cand, base)
    mask = key >= base

    g = -jnp.log(-jnp.log(u_ref[...]))
    score = jnp.where(mask, x + g, jnp.float32(_NEG))
    smax = jnp.max(score, axis=1, keepdims=True)
    pos = lax.broadcasted_iota(jnp.int32, (_RB, _CAP), 1)
    big = jnp.int32(2**30)
    argpos = jnp.min(jnp.where(score == smax, pos, big), axis=1, keepdims=True)
    gidx = jnp.sum(jnp.where(pos == argpos, idx_ref[...], 0), axis=1,
                   keepdims=True)
    rowbase = (pl.program_id(0) * _RB
               + lax.broadcasted_iota(jnp.int32, (_RB, 1), 0)) * _V

    m = jnp.max(x, axis=1, keepdims=True)
    e = jnp.where(mask, jnp.exp(x - m), jnp.float32(0.0))
    denom = jnp.sum(e, axis=1, keepdims=True)
    xs = jnp.sum(jnp.where(pos == argpos, x, jnp.float32(0.0)), axis=1,
                 keepdims=True)
    samp_ref[...] = gidx - rowbase
    conf_ref[...] = jnp.exp(xs - m) / denom


@jax.jit
def kernel(logits, u):
    cval, cidx, cu = _sc_filter(logits, u)
    samp, conf = pl.pallas_call(
        _tc_body,
        grid=(_ROWS // _RB,),
        in_specs=[
            pl.BlockSpec((_RB, _CAP), lambda i: (i, 0)),
            pl.BlockSpec((_RB, _CAP), lambda i: (i, 0)),
            pl.BlockSpec((_RB, _CAP), lambda i: (i, 0)),
        ],
        out_specs=[
            pl.BlockSpec((_RB, 1), lambda i: (i, 0)),
            pl.BlockSpec((_RB, 1), lambda i: (i, 0)),
        ],
        out_shape=[
            jax.ShapeDtypeStruct((_ROWS, 1), jnp.int32),
            jax.ShapeDtypeStruct((_ROWS, 1), jnp.float32),
        ],
    )(cval, cidx, cu)
    return samp[:, 0], conf[:, 0]


# speedup vs baseline: 1.2077x; 1.2077x over previous
import functools
import jax
import jax.numpy as jnp
from jax import lax
from jax.experimental import pallas as pl
from jax.experimental.pallas import tpu as pltpu
from jax.experimental.pallas import tpu_sc as plsc

_ROWS = 128
_V = 100000
_K = 64
_F = 2000
_C = 50
_CAP = 2048
_UCH = 20000
_SENT = 0x7FFFFF00
_RPW = 4
_RB = 8
_NEG = -3.4e38
_IMIN = -2147483648
_MANT = 0x7FFFFFFF


def _sc_body(logits_hbm, u_hbm, oval, oidx, ou,
             row_v, uch_v, kcm_v, cval_v, cidx_v, cu_v):
    wid = lax.axis_index("s") * 2 + lax.axis_index("c")
    lane = lax.iota(jnp.int32, 16)
    for rr in range(_RPW):
        r = wid * _RPW + rr
        pltpu.sync_copy(logits_hbm.at[r], row_v)

        def fill(i, _):
            sl = pl.ds(i * 16, 16)
            cval_v[sl] = jnp.full((16,), _NEG, jnp.float32)
            cu_v[sl] = jnp.full((16,), jnp.float32(0.5))
            cidx_v[sl] = jnp.full((16,), _SENT, jnp.int32)
            return 0
        lax.fori_loop(0, _CAP // 16, fill, 0)

        def fold(j, _):
            base = j * 16
            acc = row_v[pl.ds(base, 16)]
            for c in range(1, _C):
                acc = jnp.maximum(acc, row_v[pl.ds(c * _F + base, 16)])
            bits = plsc.bitcast(acc, jnp.int32)
            kcm_v[pl.ds(base, 16)] = jnp.where(bits < 0, bits ^ _MANT, bits)
            return 0
        lax.fori_loop(0, _F // 16, fold, 0)

        def count_ge(t):
            def cnt(j, acc):
                k16 = kcm_v[pl.ds(j * 16, 16)]
                return acc + jnp.where(k16 >= t, jnp.int32(1), jnp.int32(0))
            acc = lax.fori_loop(0, _F // 16, cnt, jnp.zeros((16,), jnp.int32))
            return jnp.sum(acc)

        kf = jnp.int32(_K)
        base0 = jnp.where(count_ge(jnp.int32(0)) >= kf, jnp.int32(0),
                          jnp.int32(_IMIN))

        def bit(i, b):
            cand = b | (jnp.int32(1) << (jnp.int32(30) - i))
            return jnp.where(count_ge(cand) >= kf, cand, b)
        t0k = lax.fori_loop(0, 31, bit, base0)

        tkv = jnp.full((16,), t0k, jnp.int32)
        t0f = plsc.bitcast(jnp.where(tkv < 0, tkv ^ _MANT, tkv), jnp.float32)

        nf = _UCH // _F
        cursor = jnp.int32(0)
        for cu_i in range(_V // _UCH):
            pltpu.sync_copy(u_hbm.at[r, pl.ds(cu_i * _UCH, _UCH)], uch_v)

            def scan_v(v, cur):
                kv = kcm_v[pl.ds(v * 16, 16)]
                hit = jnp.max(plsc.all_reduce_population_count(kv >= t0k))

                def do_hit(cur2):
                    def inner(jj, cur3):
                        pos = (cu_i * nf + jj) * _F + v * 16
                        x16 = row_v[pl.ds(pos, 16)]
                        m = x16 >= t0f
                        npass = jnp.max(plsc.all_reduce_population_count(m))
                        off = jnp.minimum(cur3, jnp.int32(_CAP - 16))
                        plsc.store_compressed(cval_v.at[pl.ds(off, 16)], x16,
                                              mask=m)
                        u16 = uch_v[pl.ds(pos - cu_i * _UCH, 16)]
                        plsc.store_compressed(cu_v.at[pl.ds(off, 16)], u16,
                                              mask=m)
                        gi = lane + (r * _V + pos)
                        plsc.store_compressed(cidx_v.at[pl.ds(off, 16)], gi,
                                              mask=m)
                        return cur3 + npass
                    return lax.fori_loop(0, nf, inner, cur2)

                return lax.cond(hit > 0, do_hit, lambda cur2: cur2, cur)
            cursor = lax.fori_loop(0, _F // 16, scan_v, cursor)

        pltpu.sync_copy(cval_v, oval.at[r])
        pltpu.sync_copy(cidx_v, oidx.at[r])
        pltpu.sync_copy(cu_v, ou.at[r])
    plsc.subcore_barrier()


def _sc_filter(logits, u):
    mesh = plsc.VectorSubcoreMesh(core_axis_name="c", subcore_axis_name="s",
                                  num_cores=2, num_subcores=16)
    return pl.kernel(
        _sc_body,
        out_type=[
            jax.ShapeDtypeStruct((_ROWS, _CAP), jnp.float32),
            jax.ShapeDtypeStruct((_ROWS, _CAP), jnp.int32),
            jax.ShapeDtypeStruct((_ROWS, _CAP), jnp.float32),
        ],
        mesh=mesh,
        scratch_types=[
            pltpu.VMEM((_V,), jnp.float32),
            pltpu.VMEM((_UCH,), jnp.float32),
            pltpu.VMEM((_F,), jnp.int32),
            pltpu.VMEM((_CAP,), jnp.float32),
            pltpu.VMEM((_CAP,), jnp.int32),
            pltpu.VMEM((_CAP,), jnp.float32),
        ],
        compiler_params=pltpu.CompilerParams(use_tc_tiling_on_sc=False,
                                             needs_layout_passes=False,
                                             has_side_effects=True),
    )(logits, u)


def _tc_body(val_ref, idx_ref, u_ref, samp_ref, conf_ref):
    x = val_ref[...]
    bits = lax.bitcast_convert_type(x, jnp.int32)
    key = jnp.where(bits < 0, bits ^ _MANT, bits)
    kf = jnp.float32(_K)

    def count_ge(t):
        return jnp.sum((key >= t).astype(jnp.float32), axis=1, keepdims=True)

    neg = jnp.full((_RB, 1), _IMIN, jnp.int32)
    zero = jnp.zeros((_RB, 1), jnp.int32)
    base = jnp.where(count_ge(zero) >= kf, zero, neg)
    for b in range(30, -1, -1):
        cand = base | jnp.int32(1 << b)
        base = jnp.where(count_ge(cand) >= kf, cand, base)
    mask = key >= base

    g = -jnp.log(-jnp.log(u_ref[...]))
    score = jnp.where(mask, x + g, jnp.float32(_NEG))
    smax = jnp.max(score, axis=1, keepdims=True)
    gall = idx_ref[...]
    gsel = jnp.min(jnp.where(score == smax, gall, jnp.int32(_SENT)), axis=1,
                   keepdims=True)
    eqg = gall == gsel
    rowbase = (pl.program_id(0) * _RB
               + lax.broadcasted_iota(jnp.int32, (_RB, 1), 0)) * _V

    m = jnp.max(x, axis=1, keepdims=True)
    e = jnp.where(mask, jnp.exp(x - m), jnp.float32(0.0))
    denom = jnp.sum(e, axis=1, keepdims=True)
    xs = jnp.sum(jnp.where(eqg, x, jnp.float32(0.0)), axis=1, keepdims=True)
    samp_ref[...] = gsel - rowbase
    conf_ref[...] = jnp.exp(xs - m) / denom


@jax.jit
def kernel(logits, u):
    cval, cidx, cu = _sc_filter(logits, u)
    samp, conf = pl.pallas_call(
        _tc_body,
        grid=(_ROWS // _RB,),
        in_specs=[
            pl.BlockSpec((_RB, _CAP), lambda i: (i, 0)),
            pl.BlockSpec((_RB, _CAP), lambda i: (i, 0)),
            pl.BlockSpec((_RB, _CAP), lambda i: (i, 0)),
        ],
        out_specs=[
            pl.BlockSpec((_RB, 1), lambda i: (i, 0)),
            pl.BlockSpec((_RB, 1), lambda i: (i, 0)),
        ],
        out_shape=[
            jax.ShapeDtypeStruct((_ROWS, 1), jnp.int32),
            jax.ShapeDtypeStruct((_ROWS, 1), jnp.float32),
        ],
    )(cval, cidx, cu)
    return samp[:, 0], conf[:, 0]
